# baseline (speedup 1.0000x reference)
import jax
import jax.numpy as jnp
from jax.experimental import pallas as pl
from jax.experimental.pallas import tpu as pltpu

N3 = 300000
N = N3 // 3
D = 128
B = 5000
NZ = N // B
CK = 10


def _body(z_ref, zp_ref, zn_ref, wz_ref, wp_ref, wn_ref, o_ref,
          s_ref, sc_ref, q_ref, cz_ref, cp_ref, cn_ref):
    g = pl.program_id(0)

    @pl.when(g == 0)
    def _init():
        s_ref[...] = jnp.zeros_like(s_ref)
        sc_ref[0] = 0.0
        sc_ref[1] = 0.0
        sc_ref[2] = 0.0
        sc_ref[3] = 0.0

    zb = z_ref[...].astype(jnp.bfloat16)
    zpb = zp_ref[...].astype(jnp.bfloat16)
    znb = zn_ref[...].astype(jnp.bfloat16)
    wz = wz_ref[0]
    wp = wp_ref[0]
    wn = wn_ref[0]

    ones_row = jnp.ones((1, D), jnp.bfloat16)

    def lanered(v, e):
        return jax.lax.dot_general(
            v, e, (((1,), (1,)), ((), ())),
            preferred_element_type=jnp.float32)

    def colsum(w, x):
        return jax.lax.dot_general(
            w, x, (((1,), (0,)), ((), ())),
            preferred_element_type=jnp.float32)

    def logsig_sum(x):
        return jnp.sum(jnp.minimum(x, 0.0) - jnp.log1p(jnp.exp(-jnp.abs(x))))

    @pl.when(g < NZ)
    def _phase_a():
        dp = lanered(ones_row, zb * zpb)
        dn = lanered(ones_row, zb * znb)
        sc_ref[0] += logsig_sum(dp)
        sc_ref[1] += logsig_sum(-dn)
        s_ref[...] += (colsum(wz.astype(jnp.bfloat16), zb)
                       + colsum(wp.astype(jnp.bfloat16), zpb)
                       + colsum(wn.astype(jnp.bfloat16), znb))
        sc_ref[2] += jnp.sum(wz) + jnp.sum(wp) + jnp.sum(wn)
        q_ref[0, g] = lanered(ones_row, zb * zb)
        q_ref[1, g] = lanered(ones_row, zpb * zpb)
        q_ref[2, g] = lanered(ones_row, znb * znb)

        @pl.when(g < CK)
        def _fill_cache():
            cz_ref[g] = zb
            cp_ref[g] = zpb
            cn_ref[g] = znb

    @pl.when(g >= NZ)
    def _phase_b():
        j = g - NZ
        mu = s_ref[...] / jnp.maximum(sc_ref[2], 1.0)
        m = jnp.sum(mu * mu)
        mu2b = (mu * -2.0).astype(jnp.bfloat16)

        def contrib(t, xb, w):
            r = lanered(mu2b, xb)
            return jnp.sum(jnp.sqrt(jnp.maximum(w * (q_ref[t, j] + r + m),
                                                0.0)))

        @pl.when(j < CK)
        def _from_cache():
            sc_ref[3] += (contrib(0, cz_ref[j], wz)
                          + contrib(1, cp_ref[j], wp)
                          + contrib(2, cn_ref[j], wn))

        @pl.when(j >= CK)
        def _from_stream():
            sc_ref[3] += (contrib(0, zb, wz) + contrib(1, zpb, wp)
                          + contrib(2, znb, wn))

    @pl.when(g == 2 * NZ - 1)
    def _fin():
        sig = 1.0 / (1.0 + jnp.exp(-sc_ref[3]))
        total = -(sc_ref[0] + sc_ref[1]) / N + sig - 0.5
        o_ref[...] = jnp.full((1, 1), total, dtype=jnp.float32)


def kernel(out, mask):
    w = mask.astype(jnp.float32).reshape(3 * NZ, 1, B)

    def omap(t):
        def f(g):
            j = g - NZ
            idx = jnp.where(g < NZ, g, jnp.where(j < CK, NZ - 1, j))
            return (t * NZ + idx, 0)
        return f

    def wmap(t):
        return lambda g: (t * NZ + g % NZ, 0, 0)

    res = pl.pallas_call(
        _body,
        grid=(2 * NZ,),
        in_specs=[
            pl.BlockSpec((B, D), omap(0)),
            pl.BlockSpec((B, D), omap(1)),
            pl.BlockSpec((B, D), omap(2)),
            pl.BlockSpec((1, 1, B), wmap(0)),
            pl.BlockSpec((1, 1, B), wmap(1)),
            pl.BlockSpec((1, 1, B), wmap(2)),
        ],
        out_specs=pl.BlockSpec((1, 1), lambda g: (0, 0)),
        out_shape=jax.ShapeDtypeStruct((1, 1), jnp.float32),
        scratch_shapes=[
            pltpu.VMEM((1, D), jnp.float32),
            pltpu.SMEM((4,), jnp.float32),
            pltpu.VMEM((3, NZ, 1, B), jnp.float32),
            pltpu.VMEM((CK, B, D), jnp.bfloat16),
            pltpu.VMEM((CK, B, D), jnp.bfloat16),
            pltpu.VMEM((CK, B, D), jnp.bfloat16),
        ],
        compiler_params=pltpu.CompilerParams(
            dimension_semantics=("arbitrary",),
        ),
    )(out, out, out, w, w, w)
    return res[0, 0]


# speedup vs baseline: 1.0506x; 1.0506x over previous
import jax
import jax.numpy as jnp
from jax.experimental import pallas as pl
from jax.experimental.pallas import tpu as pltpu

N3 = 300000
N = N3 // 3
D = 128
B = 10000
NZ = N // B


def _body(z_ref, zp_ref, zn_ref, wz_ref, wp_ref, wn_ref, o_ref,
          s_ref, sc_ref, q_ref):
    g = pl.program_id(0)

    @pl.when(g == 0)
    def _init():
        s_ref[...] = jnp.zeros_like(s_ref)
        sc_ref[0] = 0.0
        sc_ref[1] = 0.0
        sc_ref[2] = 0.0
        sc_ref[3] = 0.0

    zb = z_ref[...].astype(jnp.bfloat16)
    zpb = zp_ref[...].astype(jnp.bfloat16)
    znb = zn_ref[...].astype(jnp.bfloat16)
    wz = wz_ref[0]
    wp = wp_ref[0]
    wn = wn_ref[0]

    ones_row = jnp.ones((1, D), jnp.bfloat16)

    def lanered(v, e):
        return jax.lax.dot_general(
            v, e, (((1,), (1,)), ((), ())),
            preferred_element_type=jnp.float32)

    def colsum(w, x):
        return jax.lax.dot_general(
            w, x, (((1,), (0,)), ((), ())),
            preferred_element_type=jnp.float32)

    def logsig_sum(x):
        return jnp.sum(jnp.minimum(x, 0.0) - jnp.log1p(jnp.exp(-jnp.abs(x))))

    @pl.when(g < NZ)
    def _phase_a():
        dp = lanered(ones_row, zb * zpb)
        dn = lanered(ones_row, zb * znb)
        sc_ref[0] += logsig_sum(dp)
        sc_ref[1] += logsig_sum(-dn)
        s_ref[...] += (colsum(wz.astype(jnp.bfloat16), zb)
                       + colsum(wp.astype(jnp.bfloat16), zpb)
                       + colsum(wn.astype(jnp.bfloat16), znb))
        sc_ref[2] += jnp.sum(wz) + jnp.sum(wp) + jnp.sum(wn)
        q_ref[0, g] = lanered(ones_row, zb * zb)
        q_ref[1, g] = lanered(ones_row, zpb * zpb)
        q_ref[2, g] = lanered(ones_row, znb * znb)

    @pl.when(g >= NZ)
    def _phase_b():
        j = g - NZ
        mu = s_ref[...] / jnp.maximum(sc_ref[2], 1.0)
        m = jnp.sum(mu * mu)
        mu2b = (mu * -2.0).astype(jnp.bfloat16)

        def contrib(t, xb, w):
            r = lanered(mu2b, xb)
            return jnp.sum(jnp.sqrt(jnp.maximum(w * (q_ref[t, j] + r + m),
                                                0.0)))

        sc_ref[3] += (contrib(0, zb, wz) + contrib(1, zpb, wp)
                      + contrib(2, znb, wn))

    @pl.when(g == 2 * NZ - 1)
    def _fin():
        sig = 1.0 / (1.0 + jnp.exp(-sc_ref[3]))
        total = -(sc_ref[0] + sc_ref[1]) / N + sig - 0.5
        o_ref[...] = jnp.full((1, 1), total, dtype=jnp.float32)


def kernel(out, mask):
    w = mask.astype(jnp.float32).reshape(3 * NZ, 1, B)

    def omap(t):
        return lambda g: (t * NZ + g % NZ, 0)

    def wmap(t):
        return lambda g: (t * NZ + g % NZ, 0, 0)

    res = pl.pallas_call(
        _body,
        grid=(2 * NZ,),
        in_specs=[
            pl.BlockSpec((B, D), omap(0)),
            pl.BlockSpec((B, D), omap(1)),
            pl.BlockSpec((B, D), omap(2)),
            pl.BlockSpec((1, 1, B), wmap(0)),
            pl.BlockSpec((1, 1, B), wmap(1)),
            pl.BlockSpec((1, 1, B), wmap(2)),
        ],
        out_specs=pl.BlockSpec((1, 1), lambda g: (0, 0)),
        out_shape=jax.ShapeDtypeStruct((1, 1), jnp.float32),
        scratch_shapes=[
            pltpu.VMEM((1, D), jnp.float32),
            pltpu.SMEM((4,), jnp.float32),
            pltpu.VMEM((3, NZ, 1, B), jnp.float32),
        ],
        compiler_params=pltpu.CompilerParams(
            dimension_semantics=("arbitrary",),
        ),
    )(out, out, out, w, w, w)
    return res[0, 0]


# speedup vs baseline: 1.1591x; 1.1033x over previous
import jax
import jax.numpy as jnp
from jax.experimental import pallas as pl
from jax.experimental.pallas import tpu as pltpu

N3 = 300000
N = N3 // 3
D = 128
B = 10000
NZ = N // B


def _body(z_ref, zp_ref, zn_ref, wz_ref, wp_ref, wn_ref, o_ref,
          s_ref, sc_ref):
    g = pl.program_id(0)

    @pl.when(g == 0)
    def _init():
        s_ref[...] = jnp.zeros_like(s_ref)
        sc_ref[0] = 0.0
        sc_ref[1] = 0.0
        sc_ref[2] = 0.0
        sc_ref[3] = 0.0

    wz = wz_ref[0]
    wp = wp_ref[0]
    wn = wn_ref[0]

    ones_row = jnp.ones((1, D), jnp.bfloat16)

    def lanered(v, e):
        return jax.lax.dot_general(
            v, e, (((1,), (1,)), ((), ())),
            preferred_element_type=jnp.float32)

    def colsum(w, x):
        return jax.lax.dot_general(
            w, x, (((1,), (0,)), ((), ())),
            preferred_element_type=jnp.float32)

    def logsig_sum(x):
        return jnp.sum(jnp.minimum(x, 0.0) - jnp.log1p(jnp.exp(-jnp.abs(x))))

    @pl.when(g < NZ)
    def _phase_a():
        zb = z_ref[...].astype(jnp.bfloat16)
        zpb = zp_ref[...].astype(jnp.bfloat16)
        znb = zn_ref[...].astype(jnp.bfloat16)
        dp = lanered(ones_row, zb * zpb)
        dn = lanered(ones_row, zb * znb)
        sc_ref[0] += logsig_sum(dp)
        sc_ref[1] += logsig_sum(-dn)
        s_ref[...] += (colsum(wz.astype(jnp.bfloat16), zb)
                       + colsum(wp.astype(jnp.bfloat16), zpb)
                       + colsum(wn.astype(jnp.bfloat16), znb))
        sc_ref[2] += jnp.sum(wz) + jnp.sum(wp) + jnp.sum(wn)

    @pl.when(g >= NZ)
    def _phase_b():
        mu = s_ref[...] / jnp.maximum(sc_ref[2], 1.0)
        mub = mu.astype(jnp.bfloat16)

        def contrib(x_ref, w):
            xm = x_ref[...].astype(jnp.bfloat16) - mub
            d2 = lanered(ones_row, xm * xm)
            return jnp.sum(jnp.sqrt(w * d2))

        sc_ref[3] += (contrib(z_ref, wz) + contrib(zp_ref, wp)
                      + contrib(zn_ref, wn))

    @pl.when(g == 2 * NZ - 1)
    def _fin():
        sig = 1.0 / (1.0 + jnp.exp(-sc_ref[3]))
        total = -(sc_ref[0] + sc_ref[1]) / N + sig - 0.5
        o_ref[...] = jnp.full((1, 1), total, dtype=jnp.float32)


def kernel(out, mask):
    w = mask.astype(jnp.float32).reshape(3 * NZ, 1, B)

    def omap(t):
        return lambda g: (t * NZ + g % NZ, 0)

    def wmap(t):
        return lambda g: (t * NZ + g % NZ, 0, 0)

    res = pl.pallas_call(
        _body,
        grid=(2 * NZ,),
        in_specs=[
            pl.BlockSpec((B, D), omap(0)),
            pl.BlockSpec((B, D), omap(1)),
            pl.BlockSpec((B, D), omap(2)),
            pl.BlockSpec((1, 1, B), wmap(0)),
            pl.BlockSpec((1, 1, B), wmap(1)),
            pl.BlockSpec((1, 1, B), wmap(2)),
        ],
        out_specs=pl.BlockSpec((1, 1), lambda g: (0, 0)),
        out_shape=jax.ShapeDtypeStruct((1, 1), jnp.float32),
        scratch_shapes=[
            pltpu.VMEM((1, D), jnp.float32),
            pltpu.SMEM((4,), jnp.float32),
        ],
        compiler_params=pltpu.CompilerParams(
            dimension_semantics=("arbitrary",),
        ),
    )(out, out, out, w, w, w)
    return res[0, 0]
